# baseline (speedup 1.0000x reference)
import functools

import jax
import jax.numpy as jnp
from jax import lax
from jax.experimental import pallas as pl
from jax.experimental.pallas import tpu as pltpu
from jax.experimental.pallas import tpu_sc as plsc

N = 512
E = 4096
H = 16
NC = 2
NS = 16
NW = NC * NS
SRC_PER_W = N // NW


def _sc_body(eu_hbm, ev_hbm, ea_hbm, w_hbm, b_hbm, out_hbm,
             eu_v, ev_v, ea_v, w_v, b_v, cnt_v, ptr_v, wptr_v, col_v,
             enc_v, lvl_v, bp_v, s_v, norm_v, recip_v):
    wid = lax.axis_index("s") * NC + lax.axis_index("c")
    lane = lax.iota(jnp.int32, H)
    mask0 = lane == 0
    lane01 = jnp.minimum(lane, 1)
    lane03 = jnp.minimum(lane, 3)

    def sload(ref, idx):
        return plsc.load_gather(ref, [jnp.full((H,), idx, jnp.int32)])[0]

    def sstore(ref, idx, val):
        plsc.store_scatter(ref, [jnp.full((H,), idx, jnp.int32)],
                           jnp.full((H,), val), mask=mask0)

    def row16(ref, r):
        return plsc.load_gather(ref, [r * H + lane])

    pltpu.sync_copy(eu_hbm, eu_v)
    pltpu.sync_copy(ev_hbm, ev_v)
    pltpu.sync_copy(ea_hbm, ea_v)
    pltpu.sync_copy(w_hbm, w_v)
    pltpu.sync_copy(b_hbm, b_v)

    zeros16i = jnp.zeros((H,), jnp.int32)
    ones16i = jnp.ones((H,), jnp.int32)
    zerof = jnp.zeros((H,), jnp.float32)
    minus1 = jnp.full((H,), -1, jnp.int32)

    for k in range(N // H):
        cnt_v[pl.ds(k * H, H)] = zeros16i

    def count_body(eb, _):
        u_vec = plsc.load_gather(eu_v, [eb * H + lane])
        plsc.addupdate_scatter(cnt_v, [u_vec], ones16i)
        return 0
    lax.fori_loop(0, E // H, count_body, 0)

    sstore(ptr_v, 0, jnp.int32(0))

    def prefix_body(k, carry):
        c = plsc.load_gather(cnt_v, [k * H + lane])
        incl = plsc.cumsum(c) + carry
        plsc.store_scatter(ptr_v, [k * H + 1 + lane], incl)
        plsc.store_scatter(wptr_v, [k * H + lane], incl - c)
        return incl[H - 1]
    lax.fori_loop(0, N // H, prefix_body, jnp.int32(0))

    def place_body(kb, _):
        uv = plsc.load_gather(eu_v, [kb * H + lane])
        vv = plsc.load_gather(ev_v, [kb * H + lane])
        for kk in range(H):
            u = uv[kk]
            v = vv[kk]
            p = sload(wptr_v, u)
            sstore(col_v, p, v)
            sstore(wptr_v, u, p + 1)
        return 0
    lax.fori_loop(0, E // H, place_body, 0)

    bb = b_v[:]
    wr0 = w_v[pl.ds(0 * H, H)]
    wr1 = w_v[pl.ds(1 * H, H)]
    wr2 = w_v[pl.ds(2 * H, H)]
    wr3 = w_v[pl.ds(3 * H, H)]

    def enc_body(j, _):
        av = plsc.load_gather(ea_v, [4 * j + lane03])
        row = bb + av[0] * wr0 + av[1] * wr1 + av[2] * wr2 + av[3] * wr3
        plsc.store_scatter(enc_v, [j * H + lane], row)
        return 0
    lax.fori_loop(0, N, enc_body, 0)

    def recip_body(k, _):
        d = k * H + lane
        r = 1.0 / (d + 1).astype(jnp.float32)
        plsc.store_scatter(recip_v, [d], r)
        return 0
    lax.fori_loop(0, N // H, recip_body, 0)

    def pop_node(p, e0, e1, lvec):
        nb = (e1 - e0 + (H - 1)) // H
        pvec = jnp.full((H,), p)

        def blk_body(t, _):
            base = e0 + t * H
            eidx = jnp.minimum(base + lane, E - 1)
            valid = (base + lane) < e1
            jv = plsc.load_gather(col_v, [eidx])
            jvs = jnp.where(valid, jv, 0)
            lvj = plsc.load_gather(lvl_v, [jvs])
            cand = jnp.logical_and(valid, lvj == -1)
            plsc.store_scatter(lvl_v, [jvs], lvec, mask=cand)
            plsc.store_scatter(bp_v, [jvs], pvec, mask=cand)
            return 0

        lax.fori_loop(0, nb, blk_body, 0)

    def src_body(s, _):
        i = wid * SRC_PER_W + s

        for k in range(N // H):
            lvl_v[pl.ds(k * H, H)] = minus1

        sstore(lvl_v, i, jnp.int32(0))
        plsc.store_scatter(s_v, [i * H + lane], row16(enc_v, i))

        pe_i = plsc.load_gather(ptr_v, [i + lane01])
        pop_node(i, pe_i[0], pe_i[1], jnp.full((H,), jnp.int32(1)))

        def lvl_cond(c):
            level, found = c
            return found

        def lvl_body(c):
            level, _ = c
            lvec = jnp.full((H,), level)
            lm1 = level - 1

            def scan_body(k, found):
                nodes = k * H + lane
                lv = plsc.load_gather(lvl_v, [nodes])
                fm = lv == lm1
                anyf = jnp.any(fm)

                def have(_):
                    ps = plsc.load_gather(ptr_v, [nodes])
                    pe = plsc.load_gather(ptr_v, [nodes + 1])
                    for kk in range(H):
                        @pl.when(lv[kk] == lm1)
                        def _():
                            pop_node(k * H + kk, ps[kk], pe[kk], lvec)
                    return 0

                lax.cond(anyf, have, lambda _: 0, 0)
                return jnp.logical_or(found, anyf)

            found = lax.fori_loop(0, N // H, scan_body, False)
            return level + 1, found

        lvlf, _ = lax.while_loop(lvl_cond, lvl_body, (jnp.int32(2), True))
        maxlev = lvlf - 2

        for k in range(H * N // H):
            norm_v[pl.ds(k * H, H)] = zerof
        plsc.store_scatter(norm_v, [lane * N + i], row16(enc_v, i))

        def rec_body(l, _):
            rv = jnp.full((H,), sload(recip_v, l))

            def scan_body(k, _):
                nodes = k * H + lane
                lv = plsc.load_gather(lvl_v, [nodes])
                fm = lv == l

                def have(_):
                    bpv = plsc.load_gather(bp_v, [nodes])
                    for kk in range(H):
                        @pl.when(lv[kk] == l)
                        def _():
                            j = k * H + kk
                            sj = row16(s_v, bpv[kk]) + row16(enc_v, j)
                            plsc.store_scatter(s_v, [j * H + lane], sj)
                            plsc.store_scatter(norm_v, [lane * N + j],
                                               sj * rv)
                    return 0

                return lax.cond(jnp.any(fm), have, lambda _: 0, 0)

            lax.fori_loop(0, N // H, scan_body, 0)
            return 0
        lax.fori_loop(1, maxlev + 1, rec_body, 0)

        pltpu.sync_copy(norm_v, out_hbm.at[i])
        return 0

    lax.fori_loop(0, SRC_PER_W, src_body, 0)


@jax.jit
def _launch(eu, ev, ea_flat, w_flat, b):
    mesh = plsc.VectorSubcoreMesh(core_axis_name="c", subcore_axis_name="s",
                                  num_cores=NC, num_subcores=NS)
    f = pl.kernel(
        _sc_body,
        out_type=jax.ShapeDtypeStruct((N, H * N), jnp.float32),
        mesh=mesh,
        compiler_params=pltpu.CompilerParams(needs_layout_passes=False),
        scratch_types=[
            pltpu.VMEM((E,), jnp.int32),
            pltpu.VMEM((E,), jnp.int32),
            pltpu.VMEM((4 * N,), jnp.float32),
            pltpu.VMEM((4 * H,), jnp.float32),
            pltpu.VMEM((H,), jnp.float32),
            pltpu.VMEM((N,), jnp.int32),
            pltpu.VMEM((N + 8,), jnp.int32),
            pltpu.VMEM((N,), jnp.int32),
            pltpu.VMEM((E,), jnp.int32),
            pltpu.VMEM((N * H,), jnp.float32),
            pltpu.VMEM((N,), jnp.int32),
            pltpu.VMEM((N,), jnp.int32),
            pltpu.VMEM((N * H,), jnp.float32),
            pltpu.VMEM((H * N,), jnp.float32),
            pltpu.VMEM((N,), jnp.float32),
        ],
    )
    return f(eu, ev, ea_flat, w_flat, b)


def kernel(x, edge_idx, edge_attr, W, b):
    del x
    eu = edge_idx[0]
    ev = edge_idx[1]
    ea_flat = edge_attr[:N].reshape(-1)
    w_flat = W.reshape(-1)
    out = _launch(eu, ev, ea_flat, w_flat, b)
    return jnp.transpose(out.reshape(N, H, N), (1, 0, 2))


# speedup vs baseline: 1.3847x; 1.3847x over previous
import functools

import jax
import jax.numpy as jnp
from jax import lax
from jax.experimental import pallas as pl
from jax.experimental.pallas import tpu as pltpu
from jax.experimental.pallas import tpu_sc as plsc

N = 512
E = 4096
H = 16
NC = 2
NS = 16
NW = NC * NS
SRC_PER_W = N // NW


def _sc_body(eu_hbm, ev_hbm, ea_hbm, w_hbm, b_hbm, out_hbm,
             eu_v, ev_v, ea_v, w_v, b_v, cnt_v, ptr_v, wptr_v, col_v,
             enc_v, enc_t, lvl_v, bp_v, s_t, norm_v, recip_v):
    wid = lax.axis_index("s") * NC + lax.axis_index("c")
    lane = lax.iota(jnp.int32, H)
    mask0 = lane == 0
    lane01 = jnp.minimum(lane, 1)
    lane03 = jnp.minimum(lane, 3)

    def sload(ref, idx):
        return plsc.load_gather(ref, [jnp.full((H,), idx, jnp.int32)])[0]

    def sstore(ref, idx, val):
        plsc.store_scatter(ref, [jnp.full((H,), idx, jnp.int32)],
                           jnp.full((H,), val), mask=mask0)

    def row16(ref, r):
        return plsc.load_gather(ref, [r * H + lane])

    pltpu.sync_copy(eu_hbm, eu_v)
    pltpu.sync_copy(ev_hbm, ev_v)
    pltpu.sync_copy(ea_hbm, ea_v)
    pltpu.sync_copy(w_hbm, w_v)
    pltpu.sync_copy(b_hbm, b_v)

    zeros16i = jnp.zeros((H,), jnp.int32)
    ones16i = jnp.ones((H,), jnp.int32)
    zerof = jnp.zeros((H,), jnp.float32)
    minus1 = jnp.full((H,), -1, jnp.int32)

    for k in range(N // H):
        cnt_v[pl.ds(k * H, H)] = zeros16i

    def count_body(eb, _):
        u_vec = plsc.load_gather(eu_v, [eb * H + lane])
        plsc.addupdate_scatter(cnt_v, [u_vec], ones16i)
        return 0
    lax.fori_loop(0, E // H, count_body, 0)

    sstore(ptr_v, 0, jnp.int32(0))

    def prefix_body(k, carry):
        c = plsc.load_gather(cnt_v, [k * H + lane])
        incl = plsc.cumsum(c) + carry
        plsc.store_scatter(ptr_v, [k * H + 1 + lane], incl)
        plsc.store_scatter(wptr_v, [k * H + lane], incl - c)
        return incl[H - 1]
    lax.fori_loop(0, N // H, prefix_body, jnp.int32(0))

    def place_body(kb, _):
        uv = plsc.load_gather(eu_v, [kb * H + lane])
        vv = plsc.load_gather(ev_v, [kb * H + lane])
        for kk in range(H):
            u = uv[kk]
            v = vv[kk]
            p = sload(wptr_v, u)
            sstore(col_v, p, v)
            sstore(wptr_v, u, p + 1)
        return 0
    lax.fori_loop(0, E // H, place_body, 0)

    bb = b_v[:]
    wr0 = w_v[pl.ds(0 * H, H)]
    wr1 = w_v[pl.ds(1 * H, H)]
    wr2 = w_v[pl.ds(2 * H, H)]
    wr3 = w_v[pl.ds(3 * H, H)]

    def enc_body(j, _):
        av = plsc.load_gather(ea_v, [4 * j + lane03])
        row = bb + av[0] * wr0 + av[1] * wr1 + av[2] * wr2 + av[3] * wr3
        plsc.store_scatter(enc_v, [j * H + lane], row)
        plsc.store_scatter(enc_t, [lane * N + j], row)
        return 0
    lax.fori_loop(0, N, enc_body, 0)

    for k in range(N // H):
        bp_v[pl.ds(k * H, H)] = zeros16i

    def recip_body(k, _):
        d = k * H + lane
        r = 1.0 / (d + 1).astype(jnp.float32)
        plsc.store_scatter(recip_v, [d], r)
        return 0
    lax.fori_loop(0, N // H, recip_body, 0)

    def pop_node(p, e0, e1, lvec):
        nb = (e1 - e0 + (H - 1)) // H
        pvec = jnp.full((H,), p)

        def blk_body(t, _):
            base = e0 + t * H
            eidx = jnp.minimum(base + lane, E - 1)
            valid = (base + lane) < e1
            jv = plsc.load_gather(col_v, [eidx])
            jvs = jnp.where(valid, jv, 0)
            lvj = plsc.load_gather(lvl_v, [jvs])
            cand = jnp.logical_and(valid, lvj == -1)
            plsc.store_scatter(lvl_v, [jvs], lvec, mask=cand)
            plsc.store_scatter(bp_v, [jvs], pvec, mask=cand)
            return 0

        lax.fori_loop(0, nb, blk_body, 0)

    def src_body(s, _):
        i = wid * SRC_PER_W + s

        for k in range(N // H):
            lvl_v[pl.ds(k * H, H)] = minus1

        sstore(lvl_v, i, jnp.int32(0))
        plsc.store_scatter(s_t, [lane * N + i], row16(enc_v, i))

        pe_i = plsc.load_gather(ptr_v, [i + lane01])
        pop_node(i, pe_i[0], pe_i[1], jnp.full((H,), jnp.int32(1)))

        def lvl_cond(c):
            level, found = c
            return found

        def lvl_body(c):
            level, _ = c
            lvec = jnp.full((H,), level)
            lm1 = level - 1

            def scan_body(k, found):
                nodes = k * H + lane
                lv = plsc.load_gather(lvl_v, [nodes])
                fm = lv == lm1
                anyf = jnp.any(fm)

                def have(_):
                    def pop_cond(c2):
                        fmm = c2
                        return jnp.any(fmm)

                    def pop_body(c2):
                        fmm = c2
                        lp = plsc.all_reduce_ffs(fmm)[0]
                        fmm = jnp.logical_and(fmm, lane != lp)
                        p = k * H + lp
                        pep = plsc.load_gather(ptr_v, [p + lane01])
                        pop_node(p, pep[0], pep[1], lvec)
                        return fmm

                    lax.while_loop(pop_cond, pop_body, fm)
                    return 0

                lax.cond(anyf, have, lambda _: 0, 0)
                return jnp.logical_or(found, anyf)

            found = lax.fori_loop(0, N // H, scan_body, False)
            return level + 1, found

        lvlf, _ = lax.while_loop(lvl_cond, lvl_body, (jnp.int32(2), True))
        maxlev = lvlf - 2

        for k in range(H * N // H):
            norm_v[pl.ds(k * H, H)] = zerof
        plsc.store_scatter(norm_v, [lane * N + i], row16(enc_v, i))

        def rec_body(l, _):
            rl = sload(recip_v, l)

            def scan_body(k, _):
                nodes = k * H + lane
                lv = plsc.load_gather(lvl_v, [nodes])
                fm = lv == l

                def have(_):
                    bpv = plsc.load_gather(bp_v, [nodes])
                    for h in range(H):
                        sph = plsc.load_gather(s_t, [h * N + bpv])
                        eh = plsc.load_gather(enc_t, [h * N + nodes])
                        snew = sph + eh
                        plsc.store_scatter(s_t, [h * N + nodes], snew,
                                           mask=fm)
                        plsc.store_scatter(norm_v, [h * N + nodes],
                                           snew * rl, mask=fm)
                    return 0

                return lax.cond(jnp.any(fm), have, lambda _: 0, 0)

            lax.fori_loop(0, N // H, scan_body, 0)
            return 0
        lax.fori_loop(1, maxlev + 1, rec_body, 0)

        pltpu.sync_copy(norm_v, out_hbm.at[i])
        return 0

    lax.fori_loop(0, SRC_PER_W, src_body, 0)


@jax.jit
def _launch(eu, ev, ea_flat, w_flat, b):
    mesh = plsc.VectorSubcoreMesh(core_axis_name="c", subcore_axis_name="s",
                                  num_cores=NC, num_subcores=NS)
    f = pl.kernel(
        _sc_body,
        out_type=jax.ShapeDtypeStruct((N, H * N), jnp.float32),
        mesh=mesh,
        compiler_params=pltpu.CompilerParams(needs_layout_passes=False),
        scratch_types=[
            pltpu.VMEM((E,), jnp.int32),
            pltpu.VMEM((E,), jnp.int32),
            pltpu.VMEM((4 * N,), jnp.float32),
            pltpu.VMEM((4 * H,), jnp.float32),
            pltpu.VMEM((H,), jnp.float32),
            pltpu.VMEM((N,), jnp.int32),
            pltpu.VMEM((N + 8,), jnp.int32),
            pltpu.VMEM((N,), jnp.int32),
            pltpu.VMEM((E,), jnp.int32),
            pltpu.VMEM((N * H,), jnp.float32),
            pltpu.VMEM((H * N,), jnp.float32),
            pltpu.VMEM((N,), jnp.int32),
            pltpu.VMEM((N,), jnp.int32),
            pltpu.VMEM((H * N,), jnp.float32),
            pltpu.VMEM((H * N,), jnp.float32),
            pltpu.VMEM((N,), jnp.float32),
        ],
    )
    return f(eu, ev, ea_flat, w_flat, b)


def kernel(x, edge_idx, edge_attr, W, b):
    del x
    eu = edge_idx[0]
    ev = edge_idx[1]
    ea_flat = edge_attr[:N].reshape(-1)
    w_flat = W.reshape(-1)
    out = _launch(eu, ev, ea_flat, w_flat, b)
    return jnp.transpose(out.reshape(N, H, N), (1, 0, 2))


# speedup vs baseline: 1.6207x; 1.1705x over previous
import functools

import jax
import jax.numpy as jnp
from jax import lax
from jax.experimental import pallas as pl
from jax.experimental.pallas import tpu as pltpu
from jax.experimental.pallas import tpu_sc as plsc

N = 512
E = 4096
H = 16
NC = 2
NS = 16
NW = NC * NS
SRC_PER_W = N // NW


def _sc_body(eu_hbm, ev_hbm, ea_hbm, w_hbm, b_hbm, out_hbm,
             eu_v, ev_v, ea_v, w_v, b_v, cnt_v, ptr_v, wptr_v, col_v,
             enc_v, enc_t, lvl_v, s_t, norm_v, recip_v):
    wid = lax.axis_index("s") * NC + lax.axis_index("c")
    lane = lax.iota(jnp.int32, H)
    mask0 = lane == 0
    lane01 = jnp.minimum(lane, 1)
    lane03 = jnp.minimum(lane, 3)

    def sload(ref, idx):
        return plsc.load_gather(ref, [jnp.full((H,), idx, jnp.int32)])[0]

    def sstore(ref, idx, val):
        plsc.store_scatter(ref, [jnp.full((H,), idx, jnp.int32)],
                           jnp.full((H,), val), mask=mask0)

    def row16(ref, r):
        return plsc.load_gather(ref, [r * H + lane])

    pltpu.sync_copy(eu_hbm, eu_v)
    pltpu.sync_copy(ev_hbm, ev_v)
    pltpu.sync_copy(ea_hbm, ea_v)
    pltpu.sync_copy(w_hbm, w_v)
    pltpu.sync_copy(b_hbm, b_v)

    zeros16i = jnp.zeros((H,), jnp.int32)
    ones16i = jnp.ones((H,), jnp.int32)
    zerof = jnp.zeros((H,), jnp.float32)
    minus1 = jnp.full((H,), -1, jnp.int32)

    for k in range(N // H):
        cnt_v[pl.ds(k * H, H)] = zeros16i

    def count_body(eb, _):
        u_vec = plsc.load_gather(eu_v, [eb * H + lane])
        plsc.addupdate_scatter(cnt_v, [u_vec], ones16i)
        return 0
    lax.fori_loop(0, E // H, count_body, 0)

    sstore(ptr_v, 0, jnp.int32(0))

    def prefix_body(k, carry):
        c = plsc.load_gather(cnt_v, [k * H + lane])
        incl = plsc.cumsum(c) + carry
        plsc.store_scatter(ptr_v, [k * H + 1 + lane], incl)
        plsc.store_scatter(wptr_v, [k * H + lane], incl - c)
        return incl[H - 1]
    lax.fori_loop(0, N // H, prefix_body, jnp.int32(0))

    def place_body(kb, _):
        uv = plsc.load_gather(eu_v, [kb * H + lane])
        vv = plsc.load_gather(ev_v, [kb * H + lane])
        for kk in range(H):
            u = uv[kk]
            v = vv[kk]
            p = sload(wptr_v, u)
            sstore(col_v, p, v)
            sstore(wptr_v, u, p + 1)
        return 0
    lax.fori_loop(0, E // H, place_body, 0)

    bb = b_v[:]
    wr0 = w_v[pl.ds(0 * H, H)]
    wr1 = w_v[pl.ds(1 * H, H)]
    wr2 = w_v[pl.ds(2 * H, H)]
    wr3 = w_v[pl.ds(3 * H, H)]

    def enc_body(j, _):
        av = plsc.load_gather(ea_v, [4 * j + lane03])
        row = bb + av[0] * wr0 + av[1] * wr1 + av[2] * wr2 + av[3] * wr3
        plsc.store_scatter(enc_v, [j * H + lane], row)
        plsc.store_scatter(enc_t, [lane * N + j], row)
        return 0
    lax.fori_loop(0, N, enc_body, 0)

    def recip_body(k, _):
        d = k * H + lane
        r = 1.0 / (d + 1).astype(jnp.float32)
        plsc.store_scatter(recip_v, [d], r)
        return 0
    lax.fori_loop(0, N // H, recip_body, 0)

    def pop_node(p, e0, e1, wvec):
        def blk(base):
            eidx = jnp.minimum(base + lane, E - 1)
            valid = (base + lane) < e1
            jv = plsc.load_gather(col_v, [eidx])
            lvj = plsc.load_gather(lvl_v, [jv])
            cand = jnp.logical_and(valid, lvj < 0)
            plsc.store_scatter(lvl_v, [jv], wvec, mask=cand)

        blk(e0)

        @pl.when(e1 - e0 > H)
        def _():
            def blk_body(t, _):
                blk(e0 + t * H)
                return 0
            nb = (e1 - e0 + (H - 1)) // H
            lax.fori_loop(1, nb, blk_body, 0)

    def src_body(s, _):
        i = wid * SRC_PER_W + s

        for k in range(N // H):
            lvl_v[pl.ds(k * H, H)] = minus1

        sstore(lvl_v, i, jnp.int32(0))
        plsc.store_scatter(s_t, [lane * N + i], row16(enc_v, i))

        pe_i = plsc.load_gather(ptr_v, [i + lane01])
        pop_node(i, pe_i[0], pe_i[1], jnp.full((H,), (1 << 16) + i))

        def lvl_cond(c):
            level, found = c
            return found

        def lvl_body(c):
            level, _ = c
            wbase = level * 65536
            lm1 = level - 1

            def scan_body(k, found):
                nodes = k * H + lane
                lv = plsc.load_gather(lvl_v, [nodes])
                fm = (lv >> 16) == lm1
                anyf = jnp.any(fm)

                def have(_):
                    def pop_cond(c2):
                        fmm = c2
                        return jnp.any(fmm)

                    def pop_body(c2):
                        fmm = c2
                        lp = plsc.all_reduce_ffs(fmm)[0]
                        fmm = jnp.logical_and(fmm, lane != lp)
                        p = k * H + lp
                        pep = plsc.load_gather(ptr_v, [p + lane01])
                        pop_node(p, pep[0], pep[1],
                                 jnp.full((H,), wbase + p))
                        return fmm

                    lax.while_loop(pop_cond, pop_body, fm)
                    return 0

                lax.cond(anyf, have, lambda _: 0, 0)
                return jnp.logical_or(found, anyf)

            found = lax.fori_loop(0, N // H, scan_body, False)
            return level + 1, found

        lvlf, _ = lax.while_loop(lvl_cond, lvl_body, (jnp.int32(2), True))
        maxlev = lvlf - 2

        for k in range(H * N // H):
            norm_v[pl.ds(k * H, H)] = zerof
        plsc.store_scatter(norm_v, [lane * N + i], row16(enc_v, i))

        def rec_body(l, _):
            rl = sload(recip_v, l)

            def scan_body(k, _):
                nodes = k * H + lane
                lv = plsc.load_gather(lvl_v, [nodes])
                fm = (lv >> 16) == l

                def have(_):
                    bpv = lv & 511
                    for h in range(H):
                        sph = plsc.load_gather(s_t, [h * N + bpv])
                        eh = plsc.load_gather(enc_t, [h * N + nodes])
                        snew = sph + eh
                        plsc.store_scatter(s_t, [h * N + nodes], snew,
                                           mask=fm)
                        plsc.store_scatter(norm_v, [h * N + nodes],
                                           snew * rl, mask=fm)
                    return 0

                return lax.cond(jnp.any(fm), have, lambda _: 0, 0)

            lax.fori_loop(0, N // H, scan_body, 0)
            return 0
        lax.fori_loop(1, maxlev + 1, rec_body, 0)

        pltpu.sync_copy(norm_v, out_hbm.at[i])
        return 0

    lax.fori_loop(0, SRC_PER_W, src_body, 0)


@jax.jit
def _launch(eu, ev, ea_flat, w_flat, b):
    mesh = plsc.VectorSubcoreMesh(core_axis_name="c", subcore_axis_name="s",
                                  num_cores=NC, num_subcores=NS)
    f = pl.kernel(
        _sc_body,
        out_type=jax.ShapeDtypeStruct((N, H * N), jnp.float32),
        mesh=mesh,
        compiler_params=pltpu.CompilerParams(needs_layout_passes=False),
        scratch_types=[
            pltpu.VMEM((E,), jnp.int32),
            pltpu.VMEM((E,), jnp.int32),
            pltpu.VMEM((4 * N,), jnp.float32),
            pltpu.VMEM((4 * H,), jnp.float32),
            pltpu.VMEM((H,), jnp.float32),
            pltpu.VMEM((N,), jnp.int32),
            pltpu.VMEM((N + 8,), jnp.int32),
            pltpu.VMEM((N,), jnp.int32),
            pltpu.VMEM((E,), jnp.int32),
            pltpu.VMEM((N * H,), jnp.float32),
            pltpu.VMEM((H * N,), jnp.float32),
            pltpu.VMEM((N,), jnp.int32),
            pltpu.VMEM((H * N,), jnp.float32),
            pltpu.VMEM((H * N,), jnp.float32),
            pltpu.VMEM((N,), jnp.float32),
        ],
    )
    return f(eu, ev, ea_flat, w_flat, b)


def kernel(x, edge_idx, edge_attr, W, b):
    del x
    eu = edge_idx[0]
    ev = edge_idx[1]
    ea_flat = edge_attr[:N].reshape(-1)
    w_flat = W.reshape(-1)
    out = _launch(eu, ev, ea_flat, w_flat, b)
    return jnp.transpose(out.reshape(N, H, N), (1, 0, 2))


# speedup vs baseline: 1.7694x; 1.0918x over previous
import functools

import jax
import jax.numpy as jnp
from jax import lax
from jax.experimental import pallas as pl
from jax.experimental.pallas import tpu as pltpu
from jax.experimental.pallas import tpu_sc as plsc

N = 512
E = 4096
H = 16
NC = 2
NS = 16
NW = NC * NS
SRC_PER_W = N // NW


def _sc_body(eu_hbm, ev_hbm, ea_hbm, w_hbm, b_hbm, out_hbm,
             eu_v, ev_v, ea_v, w_v, b_v, cnt_v, ptr_v, wptr_v, col_v,
             enc_v, enc_t, lvl_v, s_t, norm_v, recip_v):
    wid = lax.axis_index("s") * NC + lax.axis_index("c")
    lane = lax.iota(jnp.int32, H)
    mask0 = lane == 0
    lane01 = jnp.minimum(lane, 1)
    lane03 = jnp.minimum(lane, 3)

    def sload(ref, idx):
        return plsc.load_gather(ref, [jnp.full((H,), idx, jnp.int32)])[0]

    def sstore(ref, idx, val):
        plsc.store_scatter(ref, [jnp.full((H,), idx, jnp.int32)],
                           jnp.full((H,), val), mask=mask0)

    def row16(ref, r):
        return plsc.load_gather(ref, [r * H + lane])

    pltpu.sync_copy(eu_hbm, eu_v)
    pltpu.sync_copy(ev_hbm, ev_v)
    pltpu.sync_copy(ea_hbm, ea_v)
    pltpu.sync_copy(w_hbm, w_v)
    pltpu.sync_copy(b_hbm, b_v)

    zeros16i = jnp.zeros((H,), jnp.int32)
    ones16i = jnp.ones((H,), jnp.int32)
    zerof = jnp.zeros((H,), jnp.float32)
    minus1 = jnp.full((H,), -1, jnp.int32)

    for k in range(N // H):
        cnt_v[pl.ds(k * H, H)] = zeros16i

    def count_body(eb, _):
        u_vec = plsc.load_gather(eu_v, [eb * H + lane])
        plsc.addupdate_scatter(cnt_v, [u_vec], ones16i)
        return 0
    lax.fori_loop(0, E // H, count_body, 0)

    sstore(ptr_v, 0, jnp.int32(0))

    def prefix_body(k, carry):
        c = plsc.load_gather(cnt_v, [k * H + lane])
        incl = plsc.cumsum(c) + carry
        plsc.store_scatter(ptr_v, [k * H + 1 + lane], incl)
        plsc.store_scatter(wptr_v, [k * H + lane], incl - c)
        return incl[H - 1]
    lax.fori_loop(0, N // H, prefix_body, jnp.int32(0))

    def place_body(kb, _):
        uv = plsc.load_gather(eu_v, [kb * H + lane])
        vv = plsc.load_gather(ev_v, [kb * H + lane])
        for kk in range(H):
            u = uv[kk]
            v = vv[kk]
            p = sload(wptr_v, u)
            sstore(col_v, p, v)
            sstore(wptr_v, u, p + 1)
        return 0
    lax.fori_loop(0, E // H, place_body, 0)

    bb = b_v[:]
    wr0 = w_v[pl.ds(0 * H, H)]
    wr1 = w_v[pl.ds(1 * H, H)]
    wr2 = w_v[pl.ds(2 * H, H)]
    wr3 = w_v[pl.ds(3 * H, H)]

    def enc_body(j, _):
        av = plsc.load_gather(ea_v, [4 * j + lane03])
        row = bb + av[0] * wr0 + av[1] * wr1 + av[2] * wr2 + av[3] * wr3
        plsc.store_scatter(enc_v, [j * H + lane], row)
        plsc.store_scatter(enc_t, [lane * N + j], row)
        return 0
    lax.fori_loop(0, N, enc_body, 0)

    def recip_body(k, _):
        d = k * H + lane
        r = 1.0 / (d + 1).astype(jnp.float32)
        plsc.store_scatter(recip_v, [d], r)
        return 0
    lax.fori_loop(0, N // H, recip_body, 0)

    def pop_node(p, e0, e1, wvec):
        def blk(base):
            eidx = jnp.minimum(base + lane, E - 1)
            valid = (base + lane) < e1
            jv = plsc.load_gather(col_v, [eidx])
            lvj = plsc.load_gather(lvl_v, [jv])
            cand = jnp.logical_and(valid, lvj < 0)
            plsc.store_scatter(lvl_v, [jv], wvec, mask=cand)

        blk(e0)

        @pl.when(e1 - e0 > H)
        def _():
            def blk_body(t, _):
                blk(e0 + t * H)
                return 0
            nb = (e1 - e0 + (H - 1)) // H
            lax.fori_loop(1, nb, blk_body, 0)

    def src_body(s, _):
        i = wid * SRC_PER_W + s

        for k in range(N // H):
            lvl_v[pl.ds(k * H, H)] = minus1

        sstore(lvl_v, i, jnp.int32(0))
        plsc.store_scatter(s_t, [lane * N + i], row16(enc_v, i))

        pe_i = plsc.load_gather(ptr_v, [i + lane01])
        pop_node(i, pe_i[0], pe_i[1], jnp.full((H,), (1 << 16) + i))

        def lvl_cond(c):
            level, found = c
            return found

        def lvl_body(c):
            level, _ = c
            wbase = level * 65536
            lm1 = level - 1

            def scan_body(k, found):
                nodes = k * H + lane
                lv = plsc.load_gather(lvl_v, [nodes])
                fm = (lv >> 16) == lm1
                anyf = jnp.any(fm)

                def have(_):
                    lp0 = plsc.all_reduce_ffs(fm)[0]
                    fm1 = jnp.logical_and(fm, lane != lp0)
                    p0 = k * H + lp0
                    pep0 = plsc.load_gather(ptr_v, [p0 + lane01])
                    jv0 = plsc.load_gather(
                        col_v, [jnp.minimum(pep0[0] + lane, E - 1)])

                    def pop_cond(c2):
                        return c2[0]

                    def pop_body(c2):
                        _, fmm, p, e0, e1, jv = c2
                        nxt = jnp.any(fmm)
                        lpn = plsc.all_reduce_ffs(fmm)[0]
                        fmm2 = jnp.logical_and(fmm, lane != lpn)
                        pn = jnp.minimum(k * H + lpn, N - 1)
                        pepn = plsc.load_gather(ptr_v, [pn + lane01])
                        jvn = plsc.load_gather(
                            col_v, [jnp.minimum(pepn[0] + lane, E - 1)])

                        valid = (e0 + lane) < e1
                        lvj = plsc.load_gather(lvl_v, [jv])
                        cand = jnp.logical_and(valid, lvj < 0)
                        plsc.store_scatter(lvl_v, [jv],
                                           jnp.full((H,), wbase + p),
                                           mask=cand)

                        @pl.when(e1 - e0 > H)
                        def _():
                            def blk_body(t, _):
                                base = e0 + t * H
                                eidx = jnp.minimum(base + lane, E - 1)
                                vld2 = (base + lane) < e1
                                jv2 = plsc.load_gather(col_v, [eidx])
                                lvj2 = plsc.load_gather(lvl_v, [jv2])
                                cnd2 = jnp.logical_and(vld2, lvj2 < 0)
                                plsc.store_scatter(
                                    lvl_v, [jv2],
                                    jnp.full((H,), wbase + p), mask=cnd2)
                                return 0
                            nb = (e1 - e0 + (H - 1)) // H
                            lax.fori_loop(1, nb, blk_body, 0)

                        return nxt, fmm2, pn, pepn[0], pepn[1], jvn

                    lax.while_loop(pop_cond, pop_body,
                                   (True, fm1, p0, pep0[0], pep0[1], jv0))
                    return 0

                lax.cond(anyf, have, lambda _: 0, 0)
                return jnp.logical_or(found, anyf)

            found = lax.fori_loop(0, N // H, scan_body, False)
            return level + 1, found

        lvlf, _ = lax.while_loop(lvl_cond, lvl_body, (jnp.int32(2), True))
        maxlev = lvlf - 2

        for k in range(H * N // H):
            norm_v[pl.ds(k * H, H)] = zerof
        plsc.store_scatter(norm_v, [lane * N + i], row16(enc_v, i))

        def rec_body(l, _):
            rl = sload(recip_v, l)

            def scan_body(k, _):
                nodes = k * H + lane
                lv = plsc.load_gather(lvl_v, [nodes])
                fm = (lv >> 16) == l

                def have(_):
                    bpv = lv & 511
                    for h in range(H):
                        sph = plsc.load_gather(s_t, [h * N + bpv])
                        eh = plsc.load_gather(enc_t, [h * N + nodes])
                        snew = sph + eh
                        plsc.store_scatter(s_t, [h * N + nodes], snew,
                                           mask=fm)
                        plsc.store_scatter(norm_v, [h * N + nodes],
                                           snew * rl, mask=fm)
                    return 0

                return lax.cond(jnp.any(fm), have, lambda _: 0, 0)

            lax.fori_loop(0, N // H, scan_body, 0)
            return 0
        lax.fori_loop(1, maxlev + 1, rec_body, 0)

        pltpu.sync_copy(norm_v, out_hbm.at[i])
        return 0

    lax.fori_loop(0, SRC_PER_W, src_body, 0)


@jax.jit
def _launch(eu, ev, ea_flat, w_flat, b):
    mesh = plsc.VectorSubcoreMesh(core_axis_name="c", subcore_axis_name="s",
                                  num_cores=NC, num_subcores=NS)
    f = pl.kernel(
        _sc_body,
        out_type=jax.ShapeDtypeStruct((N, H * N), jnp.float32),
        mesh=mesh,
        compiler_params=pltpu.CompilerParams(needs_layout_passes=False),
        scratch_types=[
            pltpu.VMEM((E,), jnp.int32),
            pltpu.VMEM((E,), jnp.int32),
            pltpu.VMEM((4 * N,), jnp.float32),
            pltpu.VMEM((4 * H,), jnp.float32),
            pltpu.VMEM((H,), jnp.float32),
            pltpu.VMEM((N,), jnp.int32),
            pltpu.VMEM((N + 8,), jnp.int32),
            pltpu.VMEM((N,), jnp.int32),
            pltpu.VMEM((E,), jnp.int32),
            pltpu.VMEM((N * H,), jnp.float32),
            pltpu.VMEM((H * N,), jnp.float32),
            pltpu.VMEM((N,), jnp.int32),
            pltpu.VMEM((H * N,), jnp.float32),
            pltpu.VMEM((H * N,), jnp.float32),
            pltpu.VMEM((N,), jnp.float32),
        ],
    )
    return f(eu, ev, ea_flat, w_flat, b)


def kernel(x, edge_idx, edge_attr, W, b):
    del x
    eu = edge_idx[0]
    ev = edge_idx[1]
    ea_flat = edge_attr[:N].reshape(-1)
    w_flat = W.reshape(-1)
    out = _launch(eu, ev, ea_flat, w_flat, b)
    return jnp.transpose(out.reshape(N, H, N), (1, 0, 2))


# speedup vs baseline: 2.0171x; 1.1400x over previous
import functools

import jax
import jax.numpy as jnp
from jax import lax
from jax.experimental import pallas as pl
from jax.experimental.pallas import tpu as pltpu
from jax.experimental.pallas import tpu_sc as plsc

N = 512
E = 4096
H = 16
NC = 2
NS = 16
NW = NC * NS
SRC_PER_W = N // NW


def _sc_body(eu_hbm, ev_hbm, ea_hbm, w_hbm, b_hbm, out_hbm,
             eu_v, ev_v, ea_v, w_v, b_v, cnt_v, ptr_v, wptr_v, col_v,
             enc_v, enc_t, lvl_v, s_t, norm_v, recip_v):
    wid = lax.axis_index("s") * NC + lax.axis_index("c")
    lane = lax.iota(jnp.int32, H)
    mask0 = lane == 0
    lane01 = jnp.minimum(lane, 1)
    lane03 = jnp.minimum(lane, 3)

    def sload(ref, idx):
        return plsc.load_gather(ref, [jnp.full((H,), idx, jnp.int32)])[0]

    def sstore(ref, idx, val):
        plsc.store_scatter(ref, [jnp.full((H,), idx, jnp.int32)],
                           jnp.full((H,), val), mask=mask0)

    def row16(ref, r):
        return plsc.load_gather(ref, [r * H + lane])

    pltpu.sync_copy(eu_hbm, eu_v)
    pltpu.sync_copy(ev_hbm, ev_v)
    pltpu.sync_copy(ea_hbm, ea_v)
    pltpu.sync_copy(w_hbm, w_v)
    pltpu.sync_copy(b_hbm, b_v)

    zeros16i = jnp.zeros((H,), jnp.int32)
    ones16i = jnp.ones((H,), jnp.int32)
    zerof = jnp.zeros((H,), jnp.float32)
    minus1 = jnp.full((H,), -1, jnp.int32)

    for k in range(N // H):
        cnt_v[pl.ds(k * H, H)] = zeros16i

    def count_body(eb, _):
        u_vec = plsc.load_gather(eu_v, [eb * H + lane])
        plsc.addupdate_scatter(cnt_v, [u_vec], ones16i)
        return 0
    lax.fori_loop(0, E // H, count_body, 0)

    sstore(ptr_v, 0, jnp.int32(0))

    def prefix_body(k, carry):
        c = plsc.load_gather(cnt_v, [k * H + lane])
        incl = plsc.cumsum(c) + carry
        plsc.store_scatter(ptr_v, [k * H + 1 + lane], incl)
        plsc.store_scatter(wptr_v, [k * H + lane], incl - c)
        return incl[H - 1]
    lax.fori_loop(0, N // H, prefix_body, jnp.int32(0))

    def place_body(kb, _):
        uv = plsc.load_gather(eu_v, [kb * H + lane])
        vv = plsc.load_gather(ev_v, [kb * H + lane])
        for kk in range(H):
            u = uv[kk]
            v = vv[kk]
            p = sload(wptr_v, u)
            sstore(col_v, p, v)
            sstore(wptr_v, u, p + 1)
        return 0
    lax.fori_loop(0, E // H, place_body, 0)

    bb = b_v[:]
    wr0 = w_v[pl.ds(0 * H, H)]
    wr1 = w_v[pl.ds(1 * H, H)]
    wr2 = w_v[pl.ds(2 * H, H)]
    wr3 = w_v[pl.ds(3 * H, H)]

    def enc_body(j, _):
        av = plsc.load_gather(ea_v, [4 * j + lane03])
        row = bb + av[0] * wr0 + av[1] * wr1 + av[2] * wr2 + av[3] * wr3
        plsc.store_scatter(enc_v, [j * H + lane], row)
        plsc.store_scatter(enc_t, [lane * N + j], row)
        return 0
    lax.fori_loop(0, N, enc_body, 0)

    def recip_body(k, _):
        d = k * H + lane
        r = 1.0 / (d + 1).astype(jnp.float32)
        plsc.store_scatter(recip_v, [d], r)
        return 0
    lax.fori_loop(0, N // H, recip_body, 0)

    def pop_node(p, e0, e1, wvec):
        def blk(base):
            eidx = jnp.minimum(base + lane, E - 1)
            valid = (base + lane) < e1
            jv = plsc.load_gather(col_v, [eidx])
            lvj = plsc.load_gather(lvl_v, [jv])
            cand = jnp.logical_and(valid, lvj < 0)
            plsc.store_scatter(lvl_v, [jv], wvec, mask=cand)

        blk(e0)

        @pl.when(e1 - e0 > H)
        def _():
            def blk_body(t, _):
                blk(e0 + t * H)
                return 0
            nb = (e1 - e0 + (H - 1)) // H
            lax.fori_loop(1, nb, blk_body, 0)

    def src_body(s, _):
        i = wid * SRC_PER_W + s

        for k in range(N // H):
            lvl_v[pl.ds(k * H, H)] = minus1

        sstore(lvl_v, i, jnp.int32(0))
        plsc.store_scatter(s_t, [lane * N + i], row16(enc_v, i))

        pe_i = plsc.load_gather(ptr_v, [i + lane01])
        pop_node(i, pe_i[0], pe_i[1], jnp.full((H,), (1 << 16) + i))

        def lvl_cond(c):
            level, found = c
            return found

        def lvl_body(c):
            level, _ = c
            wbase = level * 65536
            lm1 = level - 1

            def scan_body(k, found):
                nodes = k * H + lane
                lv = plsc.load_gather(lvl_v, [nodes])
                fm = (lv >> 16) == lm1
                anyf = jnp.any(fm)

                def have(_):
                    lp0 = plsc.all_reduce_ffs(fm)[0]
                    fm1 = jnp.logical_and(fm, lane != lp0)
                    p0 = k * H + lp0
                    pep0 = plsc.load_gather(ptr_v, [p0 + lane01])
                    jv0 = plsc.load_gather(
                        col_v, [jnp.minimum(pep0[0] + lane, E - 1)])

                    def pop_cond(c2):
                        return c2[0]

                    def pop_body(c2):
                        _, fmm, p, e0, e1, jv = c2
                        nxt = jnp.any(fmm)
                        lpn = plsc.all_reduce_ffs(fmm)[0]
                        fmm2 = jnp.logical_and(fmm, lane != lpn)
                        pn = jnp.minimum(k * H + lpn, N - 1)
                        pepn = plsc.load_gather(ptr_v, [pn + lane01])
                        jvn = plsc.load_gather(
                            col_v, [jnp.minimum(pepn[0] + lane, E - 1)])

                        valid = (e0 + lane) < e1
                        lvj = plsc.load_gather(lvl_v, [jv])
                        cand = jnp.logical_and(valid, lvj < 0)
                        plsc.store_scatter(lvl_v, [jv],
                                           jnp.full((H,), wbase + p),
                                           mask=cand)

                        @pl.when(e1 - e0 > H)
                        def _():
                            def blk_body(t, _):
                                base = e0 + t * H
                                eidx = jnp.minimum(base + lane, E - 1)
                                vld2 = (base + lane) < e1
                                jv2 = plsc.load_gather(col_v, [eidx])
                                lvj2 = plsc.load_gather(lvl_v, [jv2])
                                cnd2 = jnp.logical_and(vld2, lvj2 < 0)
                                plsc.store_scatter(
                                    lvl_v, [jv2],
                                    jnp.full((H,), wbase + p), mask=cnd2)
                                return 0
                            nb = (e1 - e0 + (H - 1)) // H
                            lax.fori_loop(1, nb, blk_body, 0)

                        return nxt, fmm2, pn, pepn[0], pepn[1], jvn

                    lax.while_loop(pop_cond, pop_body,
                                   (True, fm1, p0, pep0[0], pep0[1], jv0))
                    return 0

                lax.cond(anyf, have, lambda _: 0, 0)
                return jnp.logical_or(found, anyf)

            found = lax.fori_loop(0, N // H, scan_body, False)
            return level + 1, found

        lvlf, _ = lax.while_loop(lvl_cond, lvl_body, (jnp.int32(2), True))
        maxlev = lvlf - 2

        for k in range(H * N // H):
            norm_v[pl.ds(k * H, H)] = zerof
        plsc.store_scatter(norm_v, [lane * N + i], row16(enc_v, i))

        def rec_body(l, _):
            rl = sload(recip_v, l)

            def scan_body(k, _):
                nodes = k * H + lane
                lv = plsc.load_gather(lvl_v, [nodes])
                fm = (lv >> 16) == l

                def have(_):
                    bpv = lv & 511
                    nidx = [h * N + nodes for h in range(H)]
                    sphs = [plsc.load_gather(s_t, [h * N + bpv])
                            for h in range(H)]
                    ehs = [plsc.load_gather(enc_t, [nidx[h]])
                           for h in range(H)]
                    snews = [sphs[h] + ehs[h] for h in range(H)]
                    for h in range(H):
                        plsc.store_scatter(s_t, [nidx[h]], snews[h],
                                           mask=fm)
                    for h in range(H):
                        plsc.store_scatter(norm_v, [nidx[h]],
                                           snews[h] * rl, mask=fm)
                    return 0

                return lax.cond(jnp.any(fm), have, lambda _: 0, 0)

            lax.fori_loop(0, N // H, scan_body, 0)
            return 0
        lax.fori_loop(1, maxlev + 1, rec_body, 0)

        pltpu.sync_copy(norm_v, out_hbm.at[i])
        return 0

    lax.fori_loop(0, SRC_PER_W, src_body, 0)


@jax.jit
def _launch(eu, ev, ea_flat, w_flat, b):
    mesh = plsc.VectorSubcoreMesh(core_axis_name="c", subcore_axis_name="s",
                                  num_cores=NC, num_subcores=NS)
    f = pl.kernel(
        _sc_body,
        out_type=jax.ShapeDtypeStruct((N, H * N), jnp.float32),
        mesh=mesh,
        compiler_params=pltpu.CompilerParams(needs_layout_passes=False),
        scratch_types=[
            pltpu.VMEM((E,), jnp.int32),
            pltpu.VMEM((E,), jnp.int32),
            pltpu.VMEM((4 * N,), jnp.float32),
            pltpu.VMEM((4 * H,), jnp.float32),
            pltpu.VMEM((H,), jnp.float32),
            pltpu.VMEM((N,), jnp.int32),
            pltpu.VMEM((N + 8,), jnp.int32),
            pltpu.VMEM((N,), jnp.int32),
            pltpu.VMEM((E,), jnp.int32),
            pltpu.VMEM((N * H,), jnp.float32),
            pltpu.VMEM((H * N,), jnp.float32),
            pltpu.VMEM((N,), jnp.int32),
            pltpu.VMEM((H * N,), jnp.float32),
            pltpu.VMEM((H * N,), jnp.float32),
            pltpu.VMEM((N,), jnp.float32),
        ],
    )
    return f(eu, ev, ea_flat, w_flat, b)


def kernel(x, edge_idx, edge_attr, W, b):
    del x
    eu = edge_idx[0]
    ev = edge_idx[1]
    ea_flat = edge_attr[:N].reshape(-1)
    w_flat = W.reshape(-1)
    out = _launch(eu, ev, ea_flat, w_flat, b)
    return jnp.transpose(out.reshape(N, H, N), (1, 0, 2))


# speedup vs baseline: 2.1602x; 1.0709x over previous
import functools

import jax
import jax.numpy as jnp
from jax import lax
from jax.experimental import pallas as pl
from jax.experimental.pallas import tpu as pltpu
from jax.experimental.pallas import tpu_sc as plsc

N = 512
E = 4096
H = 16
NC = 2
NS = 16
NW = NC * NS
SRC_PER_W = N // NW


def _sc_body(eu_hbm, ev_hbm, ea_hbm, w_hbm, b_hbm, out_hbm,
             eu_v, ev_v, ea_v, w_v, b_v, cnt_v, ptr_v, wptr_v, col_v,
             enc_v, enc_t, lvl_v, s_t, norm_v, recip_v):
    wid = lax.axis_index("s") * NC + lax.axis_index("c")
    lane = lax.iota(jnp.int32, H)
    mask0 = lane == 0
    lane01 = jnp.minimum(lane, 1)
    lane03 = jnp.minimum(lane, 3)

    def sload(ref, idx):
        return plsc.load_gather(ref, [jnp.full((H,), idx, jnp.int32)])[0]

    def sstore(ref, idx, val):
        plsc.store_scatter(ref, [jnp.full((H,), idx, jnp.int32)],
                           jnp.full((H,), val), mask=mask0)

    def row16(ref, r):
        return plsc.load_gather(ref, [r * H + lane])

    pltpu.sync_copy(eu_hbm, eu_v)
    pltpu.sync_copy(ev_hbm, ev_v)
    pltpu.sync_copy(ea_hbm, ea_v)
    pltpu.sync_copy(w_hbm, w_v)
    pltpu.sync_copy(b_hbm, b_v)

    zeros16i = jnp.zeros((H,), jnp.int32)
    ones16i = jnp.ones((H,), jnp.int32)
    zerof = jnp.zeros((H,), jnp.float32)
    minus1 = jnp.full((H,), -1, jnp.int32)

    for k in range(N // H):
        cnt_v[pl.ds(k * H, H)] = zeros16i

    def count_body(eb, _):
        u_vec = plsc.load_gather(eu_v, [eb * H + lane])
        plsc.addupdate_scatter(cnt_v, [u_vec], ones16i)
        return 0
    lax.fori_loop(0, E // H, count_body, 0)

    sstore(ptr_v, 0, jnp.int32(0))

    def prefix_body(k, carry):
        c = plsc.load_gather(cnt_v, [k * H + lane])
        incl = plsc.cumsum(c) + carry
        plsc.store_scatter(ptr_v, [k * H + 1 + lane], incl)
        plsc.store_scatter(wptr_v, [k * H + lane], incl - c)
        return incl[H - 1]
    lax.fori_loop(0, N // H, prefix_body, jnp.int32(0))

    def place_body(kb, _):
        uv = plsc.load_gather(eu_v, [kb * H + lane])
        vv = plsc.load_gather(ev_v, [kb * H + lane])
        occ, lastm = plsc.scan_count(uv)
        pos = plsc.load_gather(wptr_v, [uv]) + occ - 1
        plsc.store_scatter(col_v, [pos], vv)
        plsc.store_scatter(wptr_v, [uv], pos + 1, mask=lastm)
        return 0
    lax.fori_loop(0, E // H, place_body, 0)

    bb = b_v[:]
    wr0 = w_v[pl.ds(0 * H, H)]
    wr1 = w_v[pl.ds(1 * H, H)]
    wr2 = w_v[pl.ds(2 * H, H)]
    wr3 = w_v[pl.ds(3 * H, H)]

    def enc_body(j, _):
        av = plsc.load_gather(ea_v, [4 * j + lane03])
        row = bb + av[0] * wr0 + av[1] * wr1 + av[2] * wr2 + av[3] * wr3
        plsc.store_scatter(enc_v, [j * H + lane], row)
        plsc.store_scatter(enc_t, [lane * N + j], row)
        return 0
    lax.fori_loop(0, N, enc_body, 0)

    def recip_body(k, _):
        d = k * H + lane
        r = 1.0 / (d + 1).astype(jnp.float32)
        plsc.store_scatter(recip_v, [d], r)
        return 0
    lax.fori_loop(0, N // H, recip_body, 0)

    def pop_node(p, e0, e1, wvec):
        def blk(base):
            eidx = jnp.minimum(base + lane, E - 1)
            valid = (base + lane) < e1
            jv = plsc.load_gather(col_v, [eidx])
            lvj = plsc.load_gather(lvl_v, [jv])
            cand = jnp.logical_and(valid, lvj < 0)
            plsc.store_scatter(lvl_v, [jv], wvec, mask=cand)

        blk(e0)

        @pl.when(e1 - e0 > H)
        def _():
            def blk_body(t, _):
                blk(e0 + t * H)
                return 0
            nb = (e1 - e0 + (H - 1)) // H
            lax.fori_loop(1, nb, blk_body, 0)

    def src_body(s, _):
        i = wid * SRC_PER_W + s

        for k in range(N // H):
            lvl_v[pl.ds(k * H, H)] = minus1

        sstore(lvl_v, i, jnp.int32(0))
        plsc.store_scatter(s_t, [lane * N + i], row16(enc_v, i))

        pe_i = plsc.load_gather(ptr_v, [i + lane01])
        pop_node(i, pe_i[0], pe_i[1], jnp.full((H,), (1 << 16) + i))

        def lvl_cond(c):
            level, found = c
            return found

        def lvl_body(c):
            level, _ = c
            wbase = level * 65536
            lm1 = level - 1

            def scan_body(k, found):
                nodes = k * H + lane
                lv = plsc.load_gather(lvl_v, [nodes])
                fm = (lv >> 16) == lm1
                anyf = jnp.any(fm)

                def have(_):
                    lp0 = plsc.all_reduce_ffs(fm)[0]
                    fm1 = jnp.logical_and(fm, lane != lp0)
                    p0 = k * H + lp0
                    pep0 = plsc.load_gather(ptr_v, [p0 + lane01])
                    jv0 = plsc.load_gather(
                        col_v, [jnp.minimum(pep0[0] + lane, E - 1)])

                    def pop_cond(c2):
                        return c2[0]

                    def pop_body(c2):
                        _, fmm, p, e0, e1, jv = c2
                        nxt = jnp.any(fmm)
                        lpn = plsc.all_reduce_ffs(fmm)[0]
                        fmm2 = jnp.logical_and(fmm, lane != lpn)
                        pn = jnp.minimum(k * H + lpn, N - 1)
                        pepn = plsc.load_gather(ptr_v, [pn + lane01])
                        jvn = plsc.load_gather(
                            col_v, [jnp.minimum(pepn[0] + lane, E - 1)])

                        valid = (e0 + lane) < e1
                        lvj = plsc.load_gather(lvl_v, [jv])
                        cand = jnp.logical_and(valid, lvj < 0)
                        plsc.store_scatter(lvl_v, [jv],
                                           jnp.full((H,), wbase + p),
                                           mask=cand)

                        @pl.when(e1 - e0 > H)
                        def _():
                            def blk_body(t, _):
                                base = e0 + t * H
                                eidx = jnp.minimum(base + lane, E - 1)
                                vld2 = (base + lane) < e1
                                jv2 = plsc.load_gather(col_v, [eidx])
                                lvj2 = plsc.load_gather(lvl_v, [jv2])
                                cnd2 = jnp.logical_and(vld2, lvj2 < 0)
                                plsc.store_scatter(
                                    lvl_v, [jv2],
                                    jnp.full((H,), wbase + p), mask=cnd2)
                                return 0
                            nb = (e1 - e0 + (H - 1)) // H
                            lax.fori_loop(1, nb, blk_body, 0)

                        return nxt, fmm2, pn, pepn[0], pepn[1], jvn

                    lax.while_loop(pop_cond, pop_body,
                                   (True, fm1, p0, pep0[0], pep0[1], jv0))
                    return 0

                lax.cond(anyf, have, lambda _: 0, 0)
                return jnp.logical_or(found, anyf)

            found = lax.fori_loop(0, N // H, scan_body, False)
            return level + 1, found

        lvlf, _ = lax.while_loop(lvl_cond, lvl_body, (jnp.int32(2), True))
        maxlev = lvlf - 2

        for k in range(H * N // H):
            norm_v[pl.ds(k * H, H)] = zerof
        plsc.store_scatter(norm_v, [lane * N + i], row16(enc_v, i))

        def rec_body(l, _):
            rl = sload(recip_v, l)

            def scan_body(k, _):
                nodes = k * H + lane
                lv = plsc.load_gather(lvl_v, [nodes])
                fm = (lv >> 16) == l

                def have(_):
                    bpv = lv & 511
                    nidx = [h * N + nodes for h in range(H)]
                    sphs = [plsc.load_gather(s_t, [h * N + bpv])
                            for h in range(H)]
                    ehs = [plsc.load_gather(enc_t, [nidx[h]])
                           for h in range(H)]
                    snews = [sphs[h] + ehs[h] for h in range(H)]
                    for h in range(H):
                        plsc.store_scatter(s_t, [nidx[h]], snews[h],
                                           mask=fm)
                    for h in range(H):
                        plsc.store_scatter(norm_v, [nidx[h]],
                                           snews[h] * rl, mask=fm)
                    return 0

                return lax.cond(jnp.any(fm), have, lambda _: 0, 0)

            lax.fori_loop(0, N // H, scan_body, 0)
            return 0
        lax.fori_loop(1, maxlev + 1, rec_body, 0)

        pltpu.sync_copy(norm_v, out_hbm.at[i])
        return 0

    lax.fori_loop(0, SRC_PER_W, src_body, 0)


@jax.jit
def _launch(eu, ev, ea_flat, w_flat, b):
    mesh = plsc.VectorSubcoreMesh(core_axis_name="c", subcore_axis_name="s",
                                  num_cores=NC, num_subcores=NS)
    f = pl.kernel(
        _sc_body,
        out_type=jax.ShapeDtypeStruct((N, H * N), jnp.float32),
        mesh=mesh,
        compiler_params=pltpu.CompilerParams(needs_layout_passes=False),
        scratch_types=[
            pltpu.VMEM((E,), jnp.int32),
            pltpu.VMEM((E,), jnp.int32),
            pltpu.VMEM((4 * N,), jnp.float32),
            pltpu.VMEM((4 * H,), jnp.float32),
            pltpu.VMEM((H,), jnp.float32),
            pltpu.VMEM((N,), jnp.int32),
            pltpu.VMEM((N + 8,), jnp.int32),
            pltpu.VMEM((N,), jnp.int32),
            pltpu.VMEM((E,), jnp.int32),
            pltpu.VMEM((N * H,), jnp.float32),
            pltpu.VMEM((H * N,), jnp.float32),
            pltpu.VMEM((N,), jnp.int32),
            pltpu.VMEM((H * N,), jnp.float32),
            pltpu.VMEM((H * N,), jnp.float32),
            pltpu.VMEM((N,), jnp.float32),
        ],
    )
    return f(eu, ev, ea_flat, w_flat, b)


def kernel(x, edge_idx, edge_attr, W, b):
    del x
    eu = edge_idx[0]
    ev = edge_idx[1]
    ea_flat = edge_attr[:N].reshape(-1)
    w_flat = W.reshape(-1)
    out = _launch(eu, ev, ea_flat, w_flat, b)
    return jnp.transpose(out.reshape(N, H, N), (1, 0, 2))
